# Initial kernel scaffold; baseline (speedup 1.0000x reference)
import jax
import jax.numpy as jnp
from jax.experimental import pallas as pl


def kernel(x, pe_weight):
    raise NotImplementedError("write your pallas kernel here")


# speedup vs baseline: 3.2840x; 3.2840x over previous
import jax
import jax.numpy as jnp
from jax.experimental import pallas as pl


def _add_pe_kernel(x_ref, pe_ref, o_ref):
    o_ref[...] = x_ref[...] + pe_ref[...][None, :, :]


def kernel(x, pe_weight):
    batch, seq_len, d_model = x.shape
    s_blk = 512
    grid = (seq_len // s_blk,)
    pe = pe_weight[:seq_len]
    return pl.pallas_call(
        _add_pe_kernel,
        grid=grid,
        in_specs=[
            pl.BlockSpec((batch, s_blk, d_model), lambda i: (0, i, 0)),
            pl.BlockSpec((s_blk, d_model), lambda i: (i, 0)),
        ],
        out_specs=pl.BlockSpec((batch, s_blk, d_model), lambda i: (0, i, 0)),
        out_shape=jax.ShapeDtypeStruct((batch, seq_len, d_model), x.dtype),
    )(x, pe)


# speedup vs baseline: 3.2912x; 1.0022x over previous
import jax
import jax.numpy as jnp
from jax.experimental import pallas as pl
from jax.experimental.pallas import tpu as pltpu


def _add_pe_kernel(x_ref, pe_ref, o_ref):
    o_ref[...] = x_ref[...] + pe_ref[...][None, :, :]


def kernel(x, pe_weight):
    batch, seq_len, d_model = x.shape
    s_blk = 512
    grid = (seq_len // s_blk,)
    pe = pe_weight[:seq_len]
    return pl.pallas_call(
        _add_pe_kernel,
        grid=grid,
        in_specs=[
            pl.BlockSpec((batch, s_blk, d_model), lambda i: (0, i, 0)),
            pl.BlockSpec((s_blk, d_model), lambda i: (i, 0)),
        ],
        out_specs=pl.BlockSpec((batch, s_blk, d_model), lambda i: (0, i, 0)),
        out_shape=jax.ShapeDtypeStruct((batch, seq_len, d_model), x.dtype),
        compiler_params=pltpu.CompilerParams(
            dimension_semantics=("parallel",),
        ),
    )(x, pe)
